# Initial kernel scaffold; baseline (speedup 1.0000x reference)
import jax
import jax.numpy as jnp
from jax.experimental import pallas as pl


def kernel(points, x, W, gamma, beta):
    raise NotImplementedError("write your pallas kernel here")


# speedup vs baseline: 13.8227x; 13.8227x over previous
import jax
import jax.numpy as jnp
from jax.experimental import pallas as pl
from jax.experimental.pallas import tpu as pltpu

K_NEIGHBORS = 16
ALPHA = 0.2
EPS = 1e-5


def _knn_gather_kernel(points_ref, pt_ref, xt_ref, w1t_ref, wdt_ref,
                       e_ref, sums_ref, sumsq_ref, y1t_s, y2t_s):
    b = pl.program_id(0)
    t = pl.program_id(1)
    T = pt_ref.shape[1]
    N = points_ref.shape[2]
    k = K_NEIGHBORS

    @pl.when(t == 0)
    def _():
        xt = xt_ref[0]
        y1t_s[...] = jnp.dot(xt, w1t_ref[...], preferred_element_type=jnp.float32)
        y2t_s[...] = jnp.dot(xt, wdt_ref[...], preferred_element_type=jnp.float32)

    @pl.when(jnp.logical_and(b == 0, t == 0))
    def _():
        sums_ref[...] = jnp.zeros_like(sums_ref)
        sumsq_ref[...] = jnp.zeros_like(sumsq_ref)

    p_all = points_ref[0]
    p_t = pt_ref[0]
    xx_all = jnp.sum(p_all * p_all, axis=0, keepdims=True)
    xx_t = jnp.sum(p_t * p_t, axis=1, keepdims=True)
    inner = jnp.dot(p_t, p_all, preferred_element_type=jnp.float32)
    work = 2.0 * inner - xx_t - xx_all

    lane = jax.lax.broadcasted_iota(jnp.int32, (T, N), 1)
    y1t = y1t_s[...]
    y2tile = y2t_s[pl.ds(t * T, T), :]

    neg_inf = jnp.float32(-jnp.inf)
    s_acc = jnp.zeros_like(sums_ref)
    q_acc = jnp.zeros_like(sumsq_ref)
    for j in range(k):
        m = jnp.max(work, axis=1, keepdims=True)
        pos = jnp.where(work == m, lane, N)
        first = jnp.min(pos, axis=1, keepdims=True)
        onehot = lane == first
        g = jnp.dot(onehot.astype(jnp.float32), y1t,
                    preferred_element_type=jnp.float32)
        e_j = g + y2tile
        e_ref[0, j] = e_j
        s_acc = s_acc + jnp.sum(e_j, axis=0, keepdims=True)
        q_acc = q_acc + jnp.sum(e_j * e_j, axis=0, keepdims=True)
        if j < k - 1:
            work = jnp.where(onehot, neg_inf, work)

    sums_ref[...] += s_acc
    sumsq_ref[...] += q_acc


def _bn_act_mean_kernel(e_ref, sums_ref, sumsq_ref, gamma_ref, beta_ref,
                        nedges_ref, out_ref):
    k = e_ref.shape[1]
    cnt = nedges_ref[0, 0]
    mean = sums_ref[...] / cnt
    var = sumsq_ref[...] / cnt - mean * mean
    rstd = jax.lax.rsqrt(var + EPS)
    scale = gamma_ref[...] * rstd
    shift = beta_ref[...] - mean * scale

    e = e_ref[0]
    z = e * scale[0][None, None, :] + shift[0][None, None, :]
    z = jnp.where(z >= 0, z, ALPHA * z)
    out_ref[0] = jnp.sum(z, axis=0) / k


def kernel(points, x, W, gamma, beta):
    B, D, N = x.shape
    C = W.shape[0]
    k = K_NEIGHBORS
    T = 256 if N % 256 == 0 else 128

    xt = jnp.transpose(x, (0, 2, 1))
    pt = jnp.transpose(points, (0, 2, 1))
    W1 = W[:, :D]
    W2 = W[:, D:]
    w1t = jnp.transpose(W1)
    wdt = jnp.transpose(W2 - W1)

    grid = (B, N // T)
    e, sums, sumsq = pl.pallas_call(
        _knn_gather_kernel,
        grid=grid,
        in_specs=[
            pl.BlockSpec((1, points.shape[1], N), lambda b, t: (b, 0, 0)),
            pl.BlockSpec((1, T, points.shape[1]), lambda b, t: (b, t, 0)),
            pl.BlockSpec((1, N, D), lambda b, t: (b, 0, 0)),
            pl.BlockSpec((D, C), lambda b, t: (0, 0)),
            pl.BlockSpec((D, C), lambda b, t: (0, 0)),
        ],
        out_specs=[
            pl.BlockSpec((1, k, T, C), lambda b, t: (b, 0, t, 0)),
            pl.BlockSpec((1, C), lambda b, t: (0, 0)),
            pl.BlockSpec((1, C), lambda b, t: (0, 0)),
        ],
        out_shape=[
            jax.ShapeDtypeStruct((B, k, N, C), jnp.float32),
            jax.ShapeDtypeStruct((1, C), jnp.float32),
            jax.ShapeDtypeStruct((1, C), jnp.float32),
        ],
        scratch_shapes=[
            pltpu.VMEM((N, C), jnp.float32),
            pltpu.VMEM((N, C), jnp.float32),
        ],
    )(points, pt, xt, w1t, wdt)

    nedges = jnp.full((1, 1), float(B * N * k), dtype=jnp.float32)
    T2 = 512 if N % 512 == 0 else T
    out_t = pl.pallas_call(
        _bn_act_mean_kernel,
        grid=(B, N // T2),
        in_specs=[
            pl.BlockSpec((1, k, T2, C), lambda b, t: (b, 0, t, 0)),
            pl.BlockSpec((1, C), lambda b, t: (0, 0)),
            pl.BlockSpec((1, C), lambda b, t: (0, 0)),
            pl.BlockSpec((1, C), lambda b, t: (0, 0)),
            pl.BlockSpec((1, C), lambda b, t: (0, 0)),
            pl.BlockSpec((1, 1), lambda b, t: (0, 0), memory_space=pltpu.SMEM),
        ],
        out_specs=pl.BlockSpec((1, T2, C), lambda b, t: (b, t, 0)),
        out_shape=jax.ShapeDtypeStruct((B, N, C), jnp.float32),
    )(e, sums, sumsq, gamma.reshape(1, C), beta.reshape(1, C), nedges)

    return jnp.transpose(out_t, (0, 2, 1))
